# baseline (speedup 1.0000x reference)
import numpy as np
import jax
import jax.numpy as jnp
from jax.experimental import pallas as pl
from jax.experimental.pallas import tpu as pltpu

_NUM_CLASSES = 395
_K_REGIONS = 32
_K_SHOTS = 4
_C_FEAT = 96
_B = 128
_HW = 64 * 64
_F = _C_FEAT * _K_REGIONS


def _rand_fn(cs):
    return jax.vmap(
        lambda c: jax.random.normal(
            jax.random.fold_in(jax.random.key(1), c), (_F,), dtype=jnp.float32
        )
        * 0.01
    )(cs).astype(jnp.bfloat16)


def _try_eager_rand():
    try:
        return np.asarray(_rand_fn(jnp.arange(_NUM_CLASSES, dtype=jnp.int32)))
    except Exception:
        return None


_RAND = _try_eager_rand()


def _get_rand():
    if _RAND is not None:
        return jnp.asarray(_RAND)
    return _rand_fn(jnp.arange(_NUM_CLASSES, dtype=jnp.int32))


_GB = 8


def _topk_gather_body(cam_ref, fm_ref, out_ref, regions_ref):
    pid = pl.program_id(0)

    @pl.when(pid == 0)
    def _():
        val = cam_ref[...]
        col = jax.lax.broadcasted_iota(jnp.int32, (_B, _HW), 1)
        col_k = jax.lax.broadcasted_iota(jnp.int32, (_B, _K_REGIONS), 1)

        def body(j, carry):
            val, acc = carry
            m = jnp.max(val, axis=1, keepdims=True)
            idx = jnp.min(jnp.where(val == m, col, _HW), axis=1, keepdims=True)
            acc = jnp.where(col_k == j, idx, acc)
            val = jnp.where(col == idx, -jnp.inf, val)
            return val, acc

        _, acc = jax.lax.fori_loop(
            0, _K_REGIONS, body, (val, jnp.zeros((_B, _K_REGIONS), jnp.int32))
        )
        regions_ref[...] = acc

    iota_hw = jax.lax.broadcasted_iota(jnp.int32, (_HW, _K_REGIONS), 0)
    for bb in range(_GB):
        hw = regions_ref[pl.ds(pid * _GB + bb, 1)]
        onehot = (iota_hw == hw).astype(jnp.float32)
        out_ref[bb] = jnp.dot(
            fm_ref[bb], onehot, preferred_element_type=jnp.float32
        )


def _tc_topk_gather(cam2, fm3):
    feats3 = pl.pallas_call(
        _topk_gather_body,
        grid=(_B // _GB,),
        in_specs=[
            pl.BlockSpec((_B, _HW), lambda i: (0, 0)),
            pl.BlockSpec((_GB, _C_FEAT, _HW), lambda i: (i, 0, 0)),
        ],
        out_specs=pl.BlockSpec((_GB, _C_FEAT, _K_REGIONS), lambda i: (i, 0, 0)),
        out_shape=jax.ShapeDtypeStruct((_B, _C_FEAT, _K_REGIONS), jnp.float32),
        scratch_shapes=[pltpu.VMEM((_B, _K_REGIONS), jnp.int32)],
    )(cam2, fm3)
    return feats3.reshape(_B, _F)


def _mean_body(labels_ref, feat_ref, p0_ref, rand_ref, counts0_ref, out_ref):
    labels = labels_ref[...]
    cls = jax.lax.broadcasted_iota(jnp.int32, (_NUM_CLASSES, _B), 0)
    mask = (labels == cls).astype(jnp.float32)
    tri = (
        jax.lax.broadcasted_iota(jnp.int32, (_B, _B), 0)
        <= jax.lax.broadcasted_iota(jnp.int32, (_B, _B), 1)
    ).astype(jnp.float32)
    rank = jnp.dot(mask, tri, preferred_element_type=jnp.float32)
    sel = mask * (rank < _K_SHOTS + 0.5)
    n = jnp.sum(mask, axis=1, keepdims=True)
    msum = jnp.dot(sel, feat_ref[...], preferred_element_type=jnp.float32)
    denom = jnp.maximum(jnp.minimum(n, float(_K_SHOTS)), 1.0)
    mean = msum / denom
    p0 = p0_ref[...]
    rand = rand_ref[...].astype(jnp.float32)
    fallback = jnp.where(counts0_ref[...] == 0.0, rand, p0)
    bp = jnp.where(n > 0.0, mean, fallback)
    new = 0.9 * p0 + 0.1 * bp
    norm = jnp.sqrt(jnp.sum(new * new, axis=1, keepdims=True))
    out_ref[...] = new / (norm + 1e-8)


def kernel(cam, feature_map, labels, prototypes, counts):
    cam2 = cam.reshape(_B, _HW)
    features = _tc_topk_gather(cam2, feature_map.reshape(_B, _C_FEAT, _HW))

    out = pl.pallas_call(
        _mean_body,
        out_shape=jax.ShapeDtypeStruct((_NUM_CLASSES, _F), jnp.float32),
    )(
        labels.reshape(1, _B),
        features,
        prototypes[:, 0],
        _get_rand(),
        counts[:, 0:1],
    )
    return out
